# baseline (speedup 1.0000x reference)
import functools

import jax
import jax.numpy as jnp
from jax import lax
from jax.experimental import pallas as pl
from jax.experimental.pallas import tpu as pltpu
from jax.experimental.pallas import tpu_sc as plsc

T, H, W, C = 2, 224, 224, 192
NUM_HEADS = 6
HD = C // NUM_HEADS
WS = 5
R = WS // 2
NPIX = T * H * W
NROW = T * H
RB = 8
LANES = 16


def _qkv_body(x_ref, wq_ref, wk_ref, wv_ref, bq_ref, bk_ref, bv_ref,
              qt_ref, kt_ref, vt_ref):
    scale = HD ** -0.5
    dn = (((0,), (1,)), ((), ()))
    wq, wk, wv = wq_ref[...], wk_ref[...], wv_ref[...]
    for rr in range(RB):
        x = x_ref[pl.ds(rr * W, W), :]
        q = lax.dot_general(wq, x, dn, preferred_element_type=jnp.float32)
        qt_ref[rr] = (q + bq_ref[...]) * scale
        k = lax.dot_general(wk, x, dn, preferred_element_type=jnp.float32)
        kt_ref[rr] = k + bk_ref[...]
        v = lax.dot_general(wv, x, dn, preferred_element_type=jnp.float32)
        vt_ref[rr] = v + bv_ref[...]


def _proj_body(ot_ref, wp_ref, bp_ref, out_ref):
    dn = (((0,), (0,)), ((), ()))
    for rr in range(RB):
        o = ot_ref[rr]
        y = lax.dot_general(o, wp_ref[...], dn,
                            preferred_element_type=jnp.float32)
        out_ref[pl.ds(rr * W, W), :] = y + bp_ref[...]


def _qkv_call(x, Wq, Wk, Wv, bq, bk, bv, interpret=False):
    nblk = NROW // RB
    wspec = pl.BlockSpec((C, C), lambda i: (0, 0))
    bspec = pl.BlockSpec((C, 1), lambda i: (0, 0))
    ospec = pl.BlockSpec((RB, C, W), lambda i: (i, 0, 0))
    oshape = jax.ShapeDtypeStruct((NROW, C, W), jnp.float32)
    return pl.pallas_call(
        _qkv_body,
        grid=(nblk,),
        in_specs=[pl.BlockSpec((RB * W, C), lambda i: (i, 0)),
                  wspec, wspec, wspec, bspec, bspec, bspec],
        out_specs=[ospec, ospec, ospec],
        out_shape=[oshape] * 3,
        interpret=interpret,
    )(x, Wq, Wk, Wv, bq, bk, bv)


def _proj_call(ot, Wp, bp, interpret=False):
    nblk = NROW // RB
    return pl.pallas_call(
        _proj_body,
        grid=(nblk,),
        in_specs=[pl.BlockSpec((RB, C, W), lambda i: (i, 0, 0)),
                  pl.BlockSpec((C, C), lambda i: (0, 0)),
                  pl.BlockSpec((1, C), lambda i: (0, 0))],
        out_specs=pl.BlockSpec((RB * W, C), lambda i: (i, 0)),
        out_shape=jax.ShapeDtypeStruct((NPIX, C), jnp.float32),
        interpret=interpret,
    )(ot, Wp, bp)


_SC_PARAMS = pltpu.CompilerParams(use_tc_tiling_on_sc=False,
                                  needs_layout_passes=False)


@functools.lru_cache(maxsize=None)
def _build_attn(t_, h_, w_, c_, heads):
    hd = c_ // heads
    nrow = t_ * h_
    nchunk = w_ // LANES
    nc, ns = 2, 16
    nworker = nc * ns
    rows_per_w = nrow // nworker
    ring = WS + 1
    mesh = plsc.VectorSubcoreMesh(core_axis_name="c", subcore_axis_name="s",
                                  num_cores=nc, num_subcores=ns)

    cunroll = 4
    assert hd % cunroll == 0

    @functools.partial(
        pl.kernel,
        out_type=jax.ShapeDtypeStruct((nrow, c_, w_), jnp.float32),
        mesh=mesh,
        scratch_types=[
            pltpu.VMEM((ring, hd, w_ + 12), jnp.float32),
            pltpu.VMEM((ring, hd, w_ + 24), jnp.float32),
            pltpu.VMEM((2, hd, w_), jnp.float32),
            pltpu.VMEM((2, hd, w_ + 24), jnp.float32),
            pltpu.VMEM((WS * WS, w_ + 24), jnp.float32),
            pltpu.SemaphoreType.DMA,
            pltpu.SemaphoreType.DMA,
        ],
        compiler_params=_SC_PARAMS,
    )
    def attn(qt, kt, vt, ot, kbuf, vbuf, qbuf, obuf, wbuf, sem_in, sem_out):
        cid = lax.axis_index("c")
        sid = lax.axis_index("s")
        wid = cid * ns + sid
        row0 = wid * rows_per_w
        t = row0 // h_
        y0 = row0 % h_
        rbase = t * h_

        lane16 = lax.iota(jnp.int32, LANES)
        idx_l = jnp.abs(lane16 - 8) + 8
        idx_r = (w_ + 7) - jnp.abs(lane16 - 11)

        def fill(buf, slot):
            def fcc(cc, _):
                vl = plsc.load_gather(buf.at[slot, cc], [idx_l])
                buf[slot, cc, pl.ds(0, LANES)] = vl
                vr = plsc.load_gather(buf.at[slot, cc], [idx_r])
                buf[slot, cc, pl.ds(w_ - 4, LANES)] = vr
                return 0
            lax.fori_loop(0, hd, fcc, 0)

        def head_loop(n, _):
            ch0 = n * hd

            def pro(i, _):
                r = y0 - R + i

                @pl.when((r >= 0) & (r < h_))
                def _load():
                    slot = r % ring
                    pltpu.sync_copy(kt.at[rbase + r, pl.ds(ch0, hd)],
                                    kbuf.at[slot, :, pl.ds(8, w_)])
                    pltpu.sync_copy(vt.at[rbase + r, pl.ds(ch0, hd)],
                                    vbuf.at[slot, :, pl.ds(8, w_)])
                    fill(kbuf, slot)
                    fill(vbuf, slot)
                return 0

            lax.fori_loop(0, WS, pro, 0)
            pltpu.sync_copy(qt.at[rbase + y0, pl.ds(ch0, hd)], qbuf.at[0])

            def row_loop(i, _):
                y = y0 + i
                cur = i % 2
                nxt = (i + 1) % 2
                have_next = (i + 1) < rows_per_w
                rpre = y + R + 1
                pre_kv = have_next & (rpre < h_)

                @pl.when(have_next)
                def _pq():
                    pltpu.async_copy(qt.at[rbase + y + 1, pl.ds(ch0, hd)],
                                     qbuf.at[nxt], sem_in)

                @pl.when(pre_kv)
                def _pkv():
                    slot = rpre % ring
                    pltpu.async_copy(kt.at[rbase + rpre, pl.ds(ch0, hd)],
                                     kbuf.at[slot, :, pl.ds(8, w_)], sem_in)
                    pltpu.async_copy(vt.at[rbase + rpre, pl.ds(ch0, hd)],
                                     vbuf.at[slot, :, pl.ds(8, w_)], sem_in)

                slots = []
                for o in range(-R, R + 1):
                    ry = jnp.abs(y + o)
                    ry = (h_ - 1) - jnp.abs((h_ - 1) - ry)
                    slots.append(ry % ring)

                zero = jnp.zeros((LANES,), jnp.float32)

                def _tree(vals, op):
                    vals = list(vals)
                    while len(vals) > 1:
                        nv = [op(vals[k], vals[k + 1])
                              for k in range(0, len(vals) - 1, 2)]
                        if len(vals) % 2:
                            nv.append(vals[-1])
                        vals = nv
                    return vals[0]

                def do_chunk(x0):
                    def c_loop(ci, accs):
                        new = list(accs)
                        for u in range(cunroll):
                            cc = ci * cunroll + u
                            qv = qbuf[cur, cc, pl.ds(x0, LANES)]
                            j = 0
                            for dy in range(WS):
                                for dx in range(WS):
                                    kv = kbuf[slots[dy], cc,
                                              pl.ds(x0 + dx + 6, LANES)]
                                    new[j] = new[j] + qv * kv
                                    j += 1
                        return tuple(new)

                    accs = lax.fori_loop(0, hd // cunroll, c_loop,
                                         tuple(zero for _ in range(WS * WS)))

                    m = _tree(accs, jnp.maximum)
                    es = [jnp.exp(a - m) for a in accs]
                    inv = 1.0 / _tree(es, jnp.add)
                    for j in range(WS * WS):
                        wbuf[j, pl.ds(x0 + 8, LANES)] = es[j] * inv

                def chunk_loop(xc, _):
                    do_chunk(xc * LANES)
                    return 0

                lax.fori_loop(0, nchunk, chunk_loop, 0)

                def zero_loop(cc, _):
                    for kk in range((w_ + 16) // LANES):
                        obuf[cur, cc, pl.ds(kk * LANES, LANES)] = zero
                    return 0

                lax.fori_loop(0, hd, zero_loop, 0)

                nu = (w_ + 4 + LANES - 1) // LANES

                def u_loop(k, _):
                    b = 6 + k * LANES
                    ws_ = []
                    for dy in range(WS):
                        for dx in range(WS):
                            ws_.append(wbuf[dy * WS + dx,
                                            pl.ds(b + 2 - dx, LANES)])

                    def c3_loop(cc, _):
                        vvs = [vbuf[slots[dy], cc, pl.ds(b, LANES)]
                               for dy in range(WS)]
                        for dx in range(WS):
                            osum = _tree(
                                [ws_[dy * WS + dx] * vvs[dy]
                                 for dy in range(WS)], jnp.add)
                            plsc.addupdate(
                                obuf.at[cur, cc, pl.ds(b + 2 - dx, LANES)],
                                osum)
                        return 0

                    lax.fori_loop(0, hd, c3_loop, 0)
                    return 0

                lax.fori_loop(0, nu, u_loop, 0)

                @pl.when(i > 0)
                def _wstore():
                    pltpu.make_async_copy(
                        obuf.at[nxt, :, pl.ds(8, w_)],
                        ot.at[rbase + y - 1, pl.ds(ch0, hd)],
                        sem_out).wait()

                pltpu.async_copy(obuf.at[cur, :, pl.ds(8, w_)],
                                 ot.at[rbase + y, pl.ds(ch0, hd)], sem_out)

                @pl.when(have_next)
                def _wq():
                    pltpu.make_async_copy(
                        qt.at[rbase + y + 1, pl.ds(ch0, hd)], qbuf.at[nxt],
                        sem_in).wait()

                @pl.when(pre_kv)
                def _wkv():
                    slot = rpre % ring
                    pltpu.make_async_copy(
                        kt.at[rbase + rpre, pl.ds(ch0, hd)],
                        kbuf.at[slot, :, pl.ds(8, w_)], sem_in).wait()
                    pltpu.make_async_copy(
                        vt.at[rbase + rpre, pl.ds(ch0, hd)],
                        vbuf.at[slot, :, pl.ds(8, w_)], sem_in).wait()
                    fill(kbuf, slot)
                    fill(vbuf, slot)

                return 0

            lax.fori_loop(0, rows_per_w, row_loop, 0)
            pltpu.make_async_copy(
                obuf.at[(rows_per_w - 1) % 2, :, pl.ds(8, w_)],
                ot.at[rbase + y0 + rows_per_w - 1, pl.ds(ch0, hd)],
                sem_out).wait()
            return 0

        lax.fori_loop(0, heads, head_loop, 0)

    return attn


def kernel(vid, Wq, bq, Wk, bk, Wv, bv, Wp, bp):
    x = vid.reshape(NPIX, C)
    qt, kt, vt = _qkv_call(x, Wq, Wk, Wv, bq.reshape(C, 1), bk.reshape(C, 1),
                           bv.reshape(C, 1))
    attn = _build_attn(T, H, W, C, NUM_HEADS)
    ot = attn(qt, kt, vt)
    out = _proj_call(ot, Wp, bp.reshape(1, C))
    return out.reshape(T, H, W, C)


# speedup vs baseline: 1.0437x; 1.0437x over previous
import functools

import jax
import jax.numpy as jnp
from jax import lax
from jax.experimental import pallas as pl
from jax.experimental.pallas import tpu as pltpu
from jax.experimental.pallas import tpu_sc as plsc

T, H, W, C = 2, 224, 224, 192
NUM_HEADS = 6
HD = C // NUM_HEADS
WS = 5
R = WS // 2
NPIX = T * H * W
NROW = T * H
RB = 8
LANES = 16


def _qkv_body(x_ref, wq_ref, wk_ref, wv_ref, bq_ref, bk_ref, bv_ref,
              qt_ref, kt_ref, vt_ref):
    scale = HD ** -0.5
    dn = (((0,), (1,)), ((), ()))
    wq, wk, wv = wq_ref[...], wk_ref[...], wv_ref[...]
    for rr in range(RB):
        x = x_ref[pl.ds(rr * W, W), :]
        q = lax.dot_general(wq, x, dn, preferred_element_type=jnp.float32)
        qt_ref[rr] = (q + bq_ref[...]) * scale
        k = lax.dot_general(wk, x, dn, preferred_element_type=jnp.float32)
        kt_ref[rr] = k + bk_ref[...]
        v = lax.dot_general(wv, x, dn, preferred_element_type=jnp.float32)
        vt_ref[rr] = v + bv_ref[...]


def _proj_body(ot_ref, wp_ref, bp_ref, out_ref):
    dn = (((0,), (0,)), ((), ()))
    for rr in range(RB):
        o = ot_ref[rr]
        y = lax.dot_general(o, wp_ref[...], dn,
                            preferred_element_type=jnp.float32)
        out_ref[pl.ds(rr * W, W), :] = y + bp_ref[...]


def _qkv_call(x, Wq, Wk, Wv, bq, bk, bv, interpret=False):
    nblk = NROW // RB
    wspec = pl.BlockSpec((C, C), lambda i: (0, 0))
    bspec = pl.BlockSpec((C, 1), lambda i: (0, 0))
    ospec = pl.BlockSpec((RB, C, W), lambda i: (i, 0, 0))
    oshape = jax.ShapeDtypeStruct((NROW, C, W), jnp.float32)
    return pl.pallas_call(
        _qkv_body,
        grid=(nblk,),
        in_specs=[pl.BlockSpec((RB * W, C), lambda i: (i, 0)),
                  wspec, wspec, wspec, bspec, bspec, bspec],
        out_specs=[ospec, ospec, ospec],
        out_shape=[oshape] * 3,
        interpret=interpret,
    )(x, Wq, Wk, Wv, bq, bk, bv)


def _proj_call(ot, Wp, bp, interpret=False):
    nblk = NROW // RB
    return pl.pallas_call(
        _proj_body,
        grid=(nblk,),
        in_specs=[pl.BlockSpec((RB, C, W), lambda i: (i, 0, 0)),
                  pl.BlockSpec((C, C), lambda i: (0, 0)),
                  pl.BlockSpec((1, C), lambda i: (0, 0))],
        out_specs=pl.BlockSpec((RB * W, C), lambda i: (i, 0)),
        out_shape=jax.ShapeDtypeStruct((NPIX, C), jnp.float32),
        interpret=interpret,
    )(ot, Wp, bp)


_SC_PARAMS = pltpu.CompilerParams(use_tc_tiling_on_sc=False,
                                  needs_layout_passes=False)


@functools.lru_cache(maxsize=None)
def _build_attn(t_, h_, w_, c_, heads):
    hd = c_ // heads
    nrow = t_ * h_
    nchunk = w_ // LANES
    nc, ns = 2, 16
    nworker = nc * ns
    rows_per_w = nrow // nworker
    ring = WS + 1
    mesh = plsc.VectorSubcoreMesh(core_axis_name="c", subcore_axis_name="s",
                                  num_cores=nc, num_subcores=ns)

    cunroll = 2
    assert hd % cunroll == 0

    @functools.partial(
        pl.kernel,
        out_type=jax.ShapeDtypeStruct((nrow, c_, w_), jnp.float32),
        mesh=mesh,
        scratch_types=[
            pltpu.VMEM((ring, hd, w_ + 12), jnp.float32),
            pltpu.VMEM((ring, hd, w_ + 24), jnp.float32),
            pltpu.VMEM((2, hd, w_), jnp.float32),
            pltpu.VMEM((2, hd, w_ + 24), jnp.float32),
            pltpu.VMEM((WS * WS, w_ + 24), jnp.float32),
            pltpu.SemaphoreType.DMA,
            pltpu.SemaphoreType.DMA,
        ],
        compiler_params=_SC_PARAMS,
    )
    def attn(qt, kt, vt, ot, kbuf, vbuf, qbuf, obuf, wbuf, sem_in, sem_out):
        cid = lax.axis_index("c")
        sid = lax.axis_index("s")
        wid = cid * ns + sid
        row0 = wid * rows_per_w
        t = row0 // h_
        y0 = row0 % h_
        rbase = t * h_

        lane16 = lax.iota(jnp.int32, LANES)
        idx_l = jnp.abs(lane16 - 8) + 8
        idx_r = (w_ + 7) - jnp.abs(lane16 - 11)

        def fill(buf, slot):
            def fcc(cc, _):
                vl = plsc.load_gather(buf.at[slot, cc], [idx_l])
                buf[slot, cc, pl.ds(0, LANES)] = vl
                vr = plsc.load_gather(buf.at[slot, cc], [idx_r])
                buf[slot, cc, pl.ds(w_ - 4, LANES)] = vr
                return 0
            lax.fori_loop(0, hd, fcc, 0)

        def head_loop(n, _):
            ch0 = n * hd

            def pro(i, _):
                r = y0 - R + i

                @pl.when((r >= 0) & (r < h_))
                def _load():
                    slot = r % ring
                    pltpu.sync_copy(kt.at[rbase + r, pl.ds(ch0, hd)],
                                    kbuf.at[slot, :, pl.ds(8, w_)])
                    pltpu.sync_copy(vt.at[rbase + r, pl.ds(ch0, hd)],
                                    vbuf.at[slot, :, pl.ds(8, w_)])
                    fill(kbuf, slot)
                    fill(vbuf, slot)
                return 0

            lax.fori_loop(0, WS, pro, 0)
            pltpu.sync_copy(qt.at[rbase + y0, pl.ds(ch0, hd)], qbuf.at[0])

            def row_loop(i, _):
                y = y0 + i
                cur = i % 2
                nxt = (i + 1) % 2
                have_next = (i + 1) < rows_per_w
                rpre = y + R + 1
                pre_kv = have_next & (rpre < h_)

                @pl.when(have_next)
                def _pq():
                    pltpu.async_copy(qt.at[rbase + y + 1, pl.ds(ch0, hd)],
                                     qbuf.at[nxt], sem_in)

                @pl.when(pre_kv)
                def _pkv():
                    slot = rpre % ring
                    pltpu.async_copy(kt.at[rbase + rpre, pl.ds(ch0, hd)],
                                     kbuf.at[slot, :, pl.ds(8, w_)], sem_in)
                    pltpu.async_copy(vt.at[rbase + rpre, pl.ds(ch0, hd)],
                                     vbuf.at[slot, :, pl.ds(8, w_)], sem_in)

                slots = []
                for o in range(-R, R + 1):
                    ry = jnp.abs(y + o)
                    ry = (h_ - 1) - jnp.abs((h_ - 1) - ry)
                    slots.append(ry % ring)

                zero = jnp.zeros((LANES,), jnp.float32)

                def _tree(vals, op):
                    vals = list(vals)
                    while len(vals) > 1:
                        nv = [op(vals[k], vals[k + 1])
                              for k in range(0, len(vals) - 1, 2)]
                        if len(vals) % 2:
                            nv.append(vals[-1])
                        vals = nv
                    return vals[0]

                def do_chunk(x0):
                    def c_loop(ci, accs):
                        new = list(accs)
                        for u in range(cunroll):
                            cc = ci * cunroll + u
                            qv = qbuf[cur, cc, pl.ds(x0, LANES)]
                            j = 0
                            for dy in range(WS):
                                for dx in range(WS):
                                    kv = kbuf[slots[dy], cc,
                                              pl.ds(x0 + dx + 6, LANES)]
                                    new[j] = new[j] + qv * kv
                                    j += 1
                        return tuple(new)

                    accs = lax.fori_loop(0, hd // cunroll, c_loop,
                                         tuple(zero for _ in range(WS * WS)))

                    m = _tree(accs, jnp.maximum)
                    es = [jnp.exp(a - m) for a in accs]
                    inv = 1.0 / _tree(es, jnp.add)
                    for j in range(WS * WS):
                        wbuf[j, pl.ds(x0 + 8, LANES)] = es[j] * inv

                def chunk_loop(xc, _):
                    do_chunk(xc * LANES)
                    return 0

                lax.fori_loop(0, nchunk, chunk_loop, 0)

                def zero_loop(cc, _):
                    for kk in range((w_ + 16) // LANES):
                        obuf[cur, cc, pl.ds(kk * LANES, LANES)] = zero
                    return 0

                lax.fori_loop(0, hd, zero_loop, 0)

                nu = (w_ + 4 + LANES - 1) // LANES

                def u_loop(k, _):
                    b = 6 + k * LANES
                    ws_ = []
                    for dy in range(WS):
                        for dx in range(WS):
                            ws_.append(wbuf[dy * WS + dx,
                                            pl.ds(b + 2 - dx, LANES)])

                    def c3_loop(cc, _):
                        vvs = [vbuf[slots[dy], cc, pl.ds(b, LANES)]
                               for dy in range(WS)]
                        for dx in range(WS):
                            osum = _tree(
                                [ws_[dy * WS + dx] * vvs[dy]
                                 for dy in range(WS)], jnp.add)
                            plsc.addupdate(
                                obuf.at[cur, cc, pl.ds(b + 2 - dx, LANES)],
                                osum)
                        return 0

                    lax.fori_loop(0, hd, c3_loop, 0)
                    return 0

                lax.fori_loop(0, nu, u_loop, 0)

                @pl.when(i > 0)
                def _wstore():
                    pltpu.make_async_copy(
                        obuf.at[nxt, :, pl.ds(8, w_)],
                        ot.at[rbase + y - 1, pl.ds(ch0, hd)],
                        sem_out).wait()

                pltpu.async_copy(obuf.at[cur, :, pl.ds(8, w_)],
                                 ot.at[rbase + y, pl.ds(ch0, hd)], sem_out)

                @pl.when(have_next)
                def _wq():
                    pltpu.make_async_copy(
                        qt.at[rbase + y + 1, pl.ds(ch0, hd)], qbuf.at[nxt],
                        sem_in).wait()

                @pl.when(pre_kv)
                def _wkv():
                    slot = rpre % ring
                    pltpu.make_async_copy(
                        kt.at[rbase + rpre, pl.ds(ch0, hd)],
                        kbuf.at[slot, :, pl.ds(8, w_)], sem_in).wait()
                    pltpu.make_async_copy(
                        vt.at[rbase + rpre, pl.ds(ch0, hd)],
                        vbuf.at[slot, :, pl.ds(8, w_)], sem_in).wait()
                    fill(kbuf, slot)
                    fill(vbuf, slot)

                return 0

            lax.fori_loop(0, rows_per_w, row_loop, 0)
            pltpu.make_async_copy(
                obuf.at[(rows_per_w - 1) % 2, :, pl.ds(8, w_)],
                ot.at[rbase + y0 + rows_per_w - 1, pl.ds(ch0, hd)],
                sem_out).wait()
            return 0

        lax.fori_loop(0, heads, head_loop, 0)

    return attn


def kernel(vid, Wq, bq, Wk, bk, Wv, bv, Wp, bp):
    x = vid.reshape(NPIX, C)
    qt, kt, vt = _qkv_call(x, Wq, Wk, Wv, bq.reshape(C, 1), bk.reshape(C, 1),
                           bv.reshape(C, 1))
    attn = _build_attn(T, H, W, C, NUM_HEADS)
    ot = attn(qt, kt, vt)
    out = _proj_call(ot, Wp, bp.reshape(1, C))
    return out.reshape(T, H, W, C)
